# baseline (speedup 1.0000x reference)
import functools

import jax
import jax.numpy as jnp
from jax import lax
from jax.experimental import pallas as pl
from jax.experimental.pallas import tpu as pltpu
from jax.experimental.pallas import tpu_sc as plsc

N = 10000
E = 320000
D = 128

CH = 128
CPT = 80
E_PAD = 32 * CPT * CH
N_OUT = 10112
OROWS = N_OUT // 16

_SELU_ALPHA = 1.6732632423543772
_SELU_SCALE = 1.0507009873554805


def _make_sc_gather():
    mesh = plsc.VectorSubcoreMesh(core_axis_name="c", subcore_axis_name="s")

    @functools.partial(
        pl.kernel,
        mesh=mesh,
        out_type=jax.ShapeDtypeStruct((E_PAD, D), jnp.float32),
        scratch_types=[
            pltpu.VMEM((CPT, CH), jnp.int32),
            pltpu.VMEM((CH, D), jnp.float32),
            pltpu.VMEM((CH, D), jnp.float32),
            pltpu.VMEM_SHARED((N_OUT, D), jnp.float32),
            pltpu.SemaphoreType.DMA,
            pltpu.SemaphoreType.DMA,
        ],
    )
    def sc_gather(z_hbm, src_hbm, msgs_hbm, src_v, rows0, rows1, z_sh,
                  sem0, sem1):
        c = lax.axis_index("c")
        s = lax.axis_index("s")
        wid = c * 16 + s

        pltpu.sync_copy(z_hbm.at[pl.ds(s * OROWS, OROWS)],
                        z_sh.at[pl.ds(s * OROWS, OROWS)])
        plsc.subcore_barrier()

        base = wid * CPT
        pltpu.sync_copy(src_hbm.at[pl.ds(base, CPT)], src_v)

        pltpu.async_copy(z_sh.at[src_v.at[0]], rows0, sem0)

        def _pair(jj, _):
            j = jj * 2
            pltpu.async_copy(z_sh.at[src_v.at[j + 1]], rows1, sem1)
            pltpu.make_async_copy(z_sh.at[src_v.at[j]], rows0, sem0).wait()
            pltpu.sync_copy(rows0, msgs_hbm.at[pl.ds((base + j) * CH, CH)])

            @pl.when(jj < CPT // 2 - 1)
            def _():
                pltpu.async_copy(z_sh.at[src_v.at[j + 2]], rows0, sem0)
            pltpu.make_async_copy(z_sh.at[src_v.at[j + 1]], rows1, sem1).wait()
            pltpu.sync_copy(rows1, msgs_hbm.at[pl.ds((base + j + 1) * CH, CH)])
            return 0
        lax.fori_loop(0, CPT // 2, _pair, 0)

    return sc_gather


def _make_sc_scatter():
    mesh = plsc.VectorSubcoreMesh(core_axis_name="c", subcore_axis_name="s")

    @functools.partial(
        pl.kernel,
        mesh=mesh,
        out_type=jax.ShapeDtypeStruct((2, N_OUT, D), jnp.float32),
        scratch_types=[
            pltpu.VMEM((CPT, CH), jnp.int32),
            pltpu.VMEM((CH, D), jnp.float32),
            pltpu.VMEM((CH, D), jnp.float32),
            pltpu.VMEM_SHARED((N_OUT, D), jnp.float32),
            pltpu.SemaphoreType.DMA,
            pltpu.SemaphoreType.DMA,
        ],
    )
    def sc_scatter(msgs_hbm, dst_hbm, out_hbm, dst_v, rows0, rows1, agg_sh,
                   sem0, sem1):
        c = lax.axis_index("c")
        s = lax.axis_index("s")
        wid = c * 16 + s

        def _zrow(i, _):
            for j in range(D // 16):
                rows0[i, 16 * j:16 * (j + 1)] = jnp.zeros((16,), jnp.float32)
            return 0
        lax.fori_loop(0, CH, _zrow, 0)
        for k in range(OROWS // CH):
            pltpu.sync_copy(rows0, agg_sh.at[pl.ds(s * OROWS + k * CH, CH)])
        pltpu.sync_copy(rows0.at[pl.ds(0, OROWS % CH)],
                        agg_sh.at[pl.ds(s * OROWS + (OROWS // CH) * CH,
                                        OROWS % CH)])
        plsc.subcore_barrier()

        base = wid * CPT
        pltpu.sync_copy(dst_hbm.at[pl.ds(base, CPT)], dst_v)

        pltpu.async_copy(msgs_hbm.at[pl.ds(base * CH, CH)], rows0, sem0)

        def _pair(jj, _):
            j = jj * 2
            pltpu.async_copy(msgs_hbm.at[pl.ds((base + j + 1) * CH, CH)],
                             rows1, sem1)
            pltpu.make_async_copy(msgs_hbm.at[pl.ds(base * CH, CH)],
                                  rows0, sem0).wait()
            pltpu.sync_copy(rows0, agg_sh.at[dst_v.at[j]], add=True)

            @pl.when(jj < CPT // 2 - 1)
            def _():
                pltpu.async_copy(msgs_hbm.at[pl.ds((base + j + 2) * CH, CH)],
                                 rows0, sem0)
            pltpu.make_async_copy(msgs_hbm.at[pl.ds(base * CH, CH)],
                                  rows1, sem1).wait()
            pltpu.sync_copy(rows1, agg_sh.at[dst_v.at[j + 1]], add=True)
            return 0
        lax.fori_loop(0, CPT // 2, _pair, 0)
        plsc.subcore_barrier()

        pltpu.sync_copy(agg_sh.at[pl.ds(s * OROWS, OROWS)],
                        out_hbm.at[c].at[pl.ds(s * OROWS, OROWS)])

    return sc_scatter


_SC_CACHE = {}


def _sc_agg(zp, src, dst):
    if "g" not in _SC_CACHE:
        _SC_CACHE["g"] = _make_sc_gather()
        _SC_CACHE["s"] = _make_sc_scatter()
    msgs = _SC_CACHE["g"](zp, src)
    return _SC_CACHE["s"](msgs, dst)


def _selu(x):
    return _SELU_SCALE * jnp.where(
        x > 0, x, _SELU_ALPHA * (jnp.exp(x) - 1.0))


def _mlp_body(z_ref, p_ref, W1_ref, b1_ref, W2_ref, b2_ref, W3_ref, b3_ref,
              W4_ref, b4_ref, out_ref):
    h = z_ref[...] + p_ref[0] + p_ref[1]
    a = _selu(jnp.dot(h, W1_ref[...], preferred_element_type=jnp.float32)
              + b1_ref[...])
    a = jnp.dot(a, W2_ref[...], preferred_element_type=jnp.float32) + b2_ref[...]
    a = _selu(jnp.dot(a, W3_ref[...], preferred_element_type=jnp.float32)
              + b3_ref[...])
    out_ref[...] = (jnp.dot(a, W4_ref[...], preferred_element_type=jnp.float32)
                    + b4_ref[...])


def _mlp(z, partials, W1, b1, W2, b2, W3, b3, W4, b4):
    BLK = 1000
    grid = (N // BLK,)
    h3 = W3.shape[1]

    def _w(shape):
        return pl.BlockSpec(shape, lambda i: tuple(0 for _ in shape))

    return pl.pallas_call(
        _mlp_body,
        grid=grid,
        in_specs=[
            pl.BlockSpec((BLK, D), lambda i: (i, 0)),
            pl.BlockSpec((2, BLK, D), lambda i: (0, i, 0)),
            _w((D, D)), _w((1, D)),
            _w((D, D)), _w((1, D)),
            _w((D, h3)), _w((1, h3)),
            _w((h3, 1)), _w((1, 1)),
        ],
        out_specs=pl.BlockSpec((BLK, 1), lambda i: (i, 0)),
        out_shape=jax.ShapeDtypeStruct((N, 1), jnp.float32),
    )(z, partials, W1, b1.reshape(1, D), W2, b2.reshape(1, D),
      W3, b3.reshape(1, h3), W4, b4.reshape(1, 1))


def kernel(z, edge_index, batch, W1, b1, W2, b2, W3, b3, W4, b4):
    pad = E_PAD - E
    src = jnp.concatenate(
        [edge_index[0], jnp.zeros((pad,), jnp.int32)]).reshape(E_PAD // CH, CH)
    pad_dst = N + (jnp.arange(pad, dtype=jnp.int32) % (N_OUT - N))
    dst = jnp.concatenate([edge_index[1], pad_dst]).reshape(E_PAD // CH, CH)
    zp = jnp.pad(z, ((0, N_OUT - N), (0, 0)))
    partials = _sc_agg(zp, src, dst)[:, :N]
    return _mlp(z, partials, W1, b1, W2, b2, W3, b3, W4, b4)


# speedup vs baseline: 1.0409x; 1.0409x over previous
import functools

import jax
import jax.numpy as jnp
from jax import lax
from jax.experimental import pallas as pl
from jax.experimental.pallas import tpu as pltpu
from jax.experimental.pallas import tpu_sc as plsc

N = 10000
E = 320000
D = 128

CH = 128
CPT = 80
E_PAD = 32 * CPT * CH
N_OUT = 10112
OROWS = N_OUT // 16

_SELU_ALPHA = 1.6732632423543772
_SELU_SCALE = 1.0507009873554805


def _make_sc_gather():
    mesh = plsc.VectorSubcoreMesh(core_axis_name="c", subcore_axis_name="s")

    @functools.partial(
        pl.kernel,
        mesh=mesh,
        out_type=jax.ShapeDtypeStruct((E_PAD, D), jnp.float32),
        scratch_types=[
            pltpu.VMEM((CPT, CH), jnp.int32),
            pltpu.VMEM((CH, D), jnp.float32),
            pltpu.VMEM((CH, D), jnp.float32),
            pltpu.VMEM_SHARED((N_OUT, D), jnp.float32),
            pltpu.SemaphoreType.DMA,
            pltpu.SemaphoreType.DMA,
        ],
    )
    def sc_gather(z_hbm, src_hbm, msgs_hbm, src_v, rows0, rows1, z_sh,
                  sem0, sem1):
        c = lax.axis_index("c")
        s = lax.axis_index("s")
        wid = c * 16 + s

        @pl.when(s < 15)
        def _():
            pltpu.sync_copy(z_hbm.at[pl.ds(s * OROWS, OROWS)],
                            z_sh.at[pl.ds(s * OROWS, OROWS)])

        @pl.when(s == 15)
        def _():
            pltpu.sync_copy(z_hbm.at[pl.ds(15 * OROWS, N - 15 * OROWS)],
                            z_sh.at[pl.ds(15 * OROWS, N - 15 * OROWS)])
        plsc.subcore_barrier()

        base = wid * CPT
        pltpu.sync_copy(src_hbm.at[pl.ds(base, CPT)], src_v)

        pltpu.async_copy(z_sh.at[src_v.at[0]], rows0, sem0)

        def _pair(jj, _):
            j = jj * 2
            pltpu.async_copy(z_sh.at[src_v.at[j + 1]], rows1, sem1)
            pltpu.make_async_copy(z_sh.at[src_v.at[j]], rows0, sem0).wait()
            pltpu.sync_copy(rows0, msgs_hbm.at[pl.ds((base + j) * CH, CH)])

            @pl.when(jj < CPT // 2 - 1)
            def _():
                pltpu.async_copy(z_sh.at[src_v.at[j + 2]], rows0, sem0)
            pltpu.make_async_copy(z_sh.at[src_v.at[j + 1]], rows1, sem1).wait()
            pltpu.sync_copy(rows1, msgs_hbm.at[pl.ds((base + j + 1) * CH, CH)])
            return 0
        lax.fori_loop(0, CPT // 2, _pair, 0)

    return sc_gather


def _make_sc_scatter():
    mesh = plsc.VectorSubcoreMesh(core_axis_name="c", subcore_axis_name="s")

    @functools.partial(
        pl.kernel,
        mesh=mesh,
        out_type=jax.ShapeDtypeStruct((2, N_OUT, D), jnp.float32),
        scratch_types=[
            pltpu.VMEM((CPT, CH), jnp.int32),
            pltpu.VMEM((CH, D), jnp.float32),
            pltpu.VMEM((CH, D), jnp.float32),
            pltpu.VMEM_SHARED((N_OUT, D), jnp.float32),
            pltpu.SemaphoreType.DMA,
            pltpu.SemaphoreType.DMA,
        ],
    )
    def sc_scatter(msgs_hbm, dst_hbm, z_hbm, out_hbm, dst_v, rows0, rows1,
                   agg_sh, sem0, sem1):
        c = lax.axis_index("c")
        s = lax.axis_index("s")
        wid = c * 16 + s

        def _zrow(i, _):
            for j in range(D // 16):
                rows0[i, 16 * j:16 * (j + 1)] = jnp.zeros((16,), jnp.float32)
            return 0
        lax.fori_loop(0, CH, _zrow, 0)

        @pl.when((c == 0) & (s < 15))
        def _():
            pltpu.sync_copy(z_hbm.at[pl.ds(s * OROWS, OROWS)],
                            agg_sh.at[pl.ds(s * OROWS, OROWS)])

        @pl.when((c == 0) & (s == 15))
        def _():
            pltpu.sync_copy(z_hbm.at[pl.ds(15 * OROWS, N - 15 * OROWS)],
                            agg_sh.at[pl.ds(15 * OROWS, N - 15 * OROWS)])
            pltpu.sync_copy(rows0.at[pl.ds(0, N_OUT - N)],
                            agg_sh.at[pl.ds(N, N_OUT - N)])

        @pl.when(c == 1)
        def _():
            for k in range(OROWS // CH):
                pltpu.sync_copy(rows0,
                                agg_sh.at[pl.ds(s * OROWS + k * CH, CH)])
            pltpu.sync_copy(rows0.at[pl.ds(0, OROWS % CH)],
                            agg_sh.at[pl.ds(s * OROWS + (OROWS // CH) * CH,
                                            OROWS % CH)])
        plsc.subcore_barrier()

        base = wid * CPT
        pltpu.sync_copy(dst_hbm.at[pl.ds(base, CPT)], dst_v)

        pltpu.async_copy(msgs_hbm.at[pl.ds(base * CH, CH)], rows0, sem0)

        def _pair(jj, _):
            j = jj * 2
            pltpu.async_copy(msgs_hbm.at[pl.ds((base + j + 1) * CH, CH)],
                             rows1, sem1)
            pltpu.make_async_copy(msgs_hbm.at[pl.ds(base * CH, CH)],
                                  rows0, sem0).wait()
            pltpu.sync_copy(rows0, agg_sh.at[dst_v.at[j]], add=True)

            @pl.when(jj < CPT // 2 - 1)
            def _():
                pltpu.async_copy(msgs_hbm.at[pl.ds((base + j + 2) * CH, CH)],
                                 rows0, sem0)
            pltpu.make_async_copy(msgs_hbm.at[pl.ds(base * CH, CH)],
                                  rows1, sem1).wait()
            pltpu.sync_copy(rows1, agg_sh.at[dst_v.at[j + 1]], add=True)
            return 0
        lax.fori_loop(0, CPT // 2, _pair, 0)
        plsc.subcore_barrier()

        pltpu.sync_copy(agg_sh.at[pl.ds(s * OROWS, OROWS)],
                        out_hbm.at[c].at[pl.ds(s * OROWS, OROWS)])

    return sc_scatter


_SC_CACHE = {}


def _sc_agg(z, src, dst):
    if "g" not in _SC_CACHE:
        _SC_CACHE["g"] = _make_sc_gather()
        _SC_CACHE["s"] = _make_sc_scatter()
    msgs = _SC_CACHE["g"](z, src)
    return _SC_CACHE["s"](msgs, dst, z)


def _selu(x):
    return _SELU_SCALE * jnp.where(
        x > 0, x, _SELU_ALPHA * (jnp.exp(x) - 1.0))


def _mlp_body(p_ref, W1_ref, b1_ref, W2_ref, b2_ref, W3_ref, b3_ref,
              W4_ref, b4_ref, out_ref):
    h = p_ref[0] + p_ref[1]
    a = _selu(jnp.dot(h, W1_ref[...], preferred_element_type=jnp.float32)
              + b1_ref[...])
    a = jnp.dot(a, W2_ref[...], preferred_element_type=jnp.float32) + b2_ref[...]
    a = _selu(jnp.dot(a, W3_ref[...], preferred_element_type=jnp.float32)
              + b3_ref[...])
    out_ref[...] = (jnp.dot(a, W4_ref[...], preferred_element_type=jnp.float32)
                    + b4_ref[...])


def _mlp(partials, W1, b1, W2, b2, W3, b3, W4, b4):
    BLK = 2000
    grid = (N // BLK,)
    h3 = W3.shape[1]

    def _w(shape):
        return pl.BlockSpec(shape, lambda i: tuple(0 for _ in shape))

    return pl.pallas_call(
        _mlp_body,
        grid=grid,
        in_specs=[
            pl.BlockSpec((2, BLK, D), lambda i: (0, i, 0)),
            _w((D, D)), _w((1, D)),
            _w((D, D)), _w((1, D)),
            _w((D, h3)), _w((1, h3)),
            _w((h3, 1)), _w((1, 1)),
        ],
        out_specs=pl.BlockSpec((BLK, 1), lambda i: (i, 0)),
        out_shape=jax.ShapeDtypeStruct((N, 1), jnp.float32),
    )(partials, W1, b1.reshape(1, D), W2, b2.reshape(1, D),
      W3, b3.reshape(1, h3), W4, b4.reshape(1, 1))


def kernel(z, edge_index, batch, W1, b1, W2, b2, W3, b3, W4, b4):
    pad = E_PAD - E
    src = jnp.concatenate(
        [edge_index[0], jnp.zeros((pad,), jnp.int32)]).reshape(E_PAD // CH, CH)
    pad_dst = N + (jnp.arange(pad, dtype=jnp.int32) % (N_OUT - N))
    dst = jnp.concatenate([edge_index[1], pad_dst]).reshape(E_PAD // CH, CH)
    partials = _sc_agg(z, src, dst)
    return _mlp(partials, W1, b1, W2, b2, W3, b3, W4, b4)


# speedup vs baseline: 1.0728x; 1.0307x over previous
import functools

import jax
import jax.numpy as jnp
from jax import lax
from jax.experimental import pallas as pl
from jax.experimental.pallas import tpu as pltpu
from jax.experimental.pallas import tpu_sc as plsc

N = 10000
E = 320000
D = 128

CH = 128
CPT = 80
E_PAD = 32 * CPT * CH
N_OUT = 10112
OROWS = N_OUT // 16

_SELU_ALPHA = 1.6732632423543772
_SELU_SCALE = 1.0507009873554805


def _make_sc_fused():
    mesh = plsc.VectorSubcoreMesh(core_axis_name="c", subcore_axis_name="s")

    @functools.partial(
        pl.kernel,
        mesh=mesh,
        out_type=(
            jax.ShapeDtypeStruct((2, N_OUT, D), jnp.float32),
            jax.ShapeDtypeStruct((E_PAD, D), jnp.float32),
        ),
        scratch_types=[
            pltpu.VMEM((CPT, CH), jnp.int32),
            pltpu.VMEM((CH, D), jnp.float32),
            pltpu.VMEM((CH, D), jnp.float32),
            pltpu.VMEM_SHARED((N_OUT, D), jnp.float32),
            pltpu.SemaphoreType.DMA,
            pltpu.SemaphoreType.DMA,
        ],
    )
    def sc_fused(z_hbm, src_hbm, dst_hbm, out_hbm, msgs_hbm, idx_v,
                 rows0, rows1, buf_sh, sem0, sem1):
        c = lax.axis_index("c")
        s = lax.axis_index("s")
        wid = c * 16 + s
        base = wid * CPT

        @pl.when(s < 15)
        def _():
            pltpu.sync_copy(z_hbm.at[pl.ds(s * OROWS, OROWS)],
                            buf_sh.at[pl.ds(s * OROWS, OROWS)])

        @pl.when(s == 15)
        def _():
            pltpu.sync_copy(z_hbm.at[pl.ds(15 * OROWS, N - 15 * OROWS)],
                            buf_sh.at[pl.ds(15 * OROWS, N - 15 * OROWS)])
        plsc.subcore_barrier()

        pltpu.sync_copy(src_hbm.at[pl.ds(base, CPT)], idx_v)
        pltpu.async_copy(buf_sh.at[idx_v.at[0]], rows0, sem0)

        def _gpair(jj, _):
            j = jj * 2
            pltpu.async_copy(buf_sh.at[idx_v.at[j + 1]], rows1, sem1)
            pltpu.make_async_copy(buf_sh.at[idx_v.at[j]], rows0, sem0).wait()
            pltpu.sync_copy(rows0, msgs_hbm.at[pl.ds((base + j) * CH, CH)])

            @pl.when(jj < CPT // 2 - 1)
            def _():
                pltpu.async_copy(buf_sh.at[idx_v.at[j + 2]], rows0, sem0)
            pltpu.make_async_copy(buf_sh.at[idx_v.at[j + 1]], rows1,
                                  sem1).wait()
            pltpu.sync_copy(rows1, msgs_hbm.at[pl.ds((base + j + 1) * CH, CH)])
            return 0
        lax.fori_loop(0, CPT // 2, _gpair, 0)
        plsc.subcore_barrier()

        def _zrow(i, _):
            for j in range(D // 16):
                rows0[i, 16 * j:16 * (j + 1)] = jnp.zeros((16,), jnp.float32)
            return 0
        lax.fori_loop(0, CH, _zrow, 0)

        @pl.when((c == 0) & (s == 15))
        def _():
            pltpu.sync_copy(rows0.at[pl.ds(0, N_OUT - N)],
                            buf_sh.at[pl.ds(N, N_OUT - N)])

        @pl.when(c == 1)
        def _():
            for k in range(OROWS // CH):
                pltpu.sync_copy(rows0,
                                buf_sh.at[pl.ds(s * OROWS + k * CH, CH)])
            pltpu.sync_copy(rows0.at[pl.ds(0, OROWS % CH)],
                            buf_sh.at[pl.ds(s * OROWS + (OROWS // CH) * CH,
                                            OROWS % CH)])
        pltpu.sync_copy(dst_hbm.at[pl.ds(base, CPT)], idx_v)
        plsc.subcore_barrier()

        pltpu.async_copy(msgs_hbm.at[pl.ds(base * CH, CH)], rows0, sem0)

        def _spair(jj, _):
            j = jj * 2
            pltpu.async_copy(msgs_hbm.at[pl.ds((base + j + 1) * CH, CH)],
                             rows1, sem1)
            pltpu.make_async_copy(msgs_hbm.at[pl.ds(base * CH, CH)],
                                  rows0, sem0).wait()
            pltpu.sync_copy(rows0, buf_sh.at[idx_v.at[j]], add=True)

            @pl.when(jj < CPT // 2 - 1)
            def _():
                pltpu.async_copy(msgs_hbm.at[pl.ds((base + j + 2) * CH, CH)],
                                 rows0, sem0)
            pltpu.make_async_copy(msgs_hbm.at[pl.ds(base * CH, CH)],
                                  rows1, sem1).wait()
            pltpu.sync_copy(rows1, buf_sh.at[idx_v.at[j + 1]], add=True)
            return 0
        lax.fori_loop(0, CPT // 2, _spair, 0)
        plsc.subcore_barrier()

        pltpu.sync_copy(buf_sh.at[pl.ds(s * OROWS, OROWS)],
                        out_hbm.at[c].at[pl.ds(s * OROWS, OROWS)])

    return sc_fused


_SC_CACHE = {}


def _sc_agg(z, src, dst):
    if "f" not in _SC_CACHE:
        _SC_CACHE["f"] = _make_sc_fused()
    return _SC_CACHE["f"](z, src, dst)[0]


def _selu(x):
    return _SELU_SCALE * jnp.where(
        x > 0, x, _SELU_ALPHA * (jnp.exp(x) - 1.0))


def _mlp_body(p_ref, W1_ref, b1_ref, W2_ref, b2_ref, W3_ref, b3_ref,
              W4_ref, b4_ref, out_ref):
    h = p_ref[0] + p_ref[1]
    a = _selu(jnp.dot(h, W1_ref[...], preferred_element_type=jnp.float32)
              + b1_ref[...])
    a = jnp.dot(a, W2_ref[...], preferred_element_type=jnp.float32) + b2_ref[...]
    a = _selu(jnp.dot(a, W3_ref[...], preferred_element_type=jnp.float32)
              + b3_ref[...])
    out_ref[...] = (jnp.dot(a, W4_ref[...], preferred_element_type=jnp.float32)
                    + b4_ref[...])


def _mlp(partials, W1, b1, W2, b2, W3, b3, W4, b4):
    BLK = 2000
    grid = (N // BLK,)
    h3 = W3.shape[1]

    def _w(shape):
        return pl.BlockSpec(shape, lambda i: tuple(0 for _ in shape))

    return pl.pallas_call(
        _mlp_body,
        grid=grid,
        in_specs=[
            pl.BlockSpec((2, BLK, D), lambda i: (0, i, 0)),
            _w((D, D)), _w((1, D)),
            _w((D, D)), _w((1, D)),
            _w((D, h3)), _w((1, h3)),
            _w((h3, 1)), _w((1, 1)),
        ],
        out_specs=pl.BlockSpec((BLK, 1), lambda i: (i, 0)),
        out_shape=jax.ShapeDtypeStruct((N, 1), jnp.float32),
    )(partials, W1, b1.reshape(1, D), W2, b2.reshape(1, D),
      W3, b3.reshape(1, h3), W4, b4.reshape(1, 1))


def kernel(z, edge_index, batch, W1, b1, W2, b2, W3, b3, W4, b4):
    pad = E_PAD - E
    src = jnp.concatenate(
        [edge_index[0], jnp.zeros((pad,), jnp.int32)]).reshape(E_PAD // CH, CH)
    pad_dst = N + (jnp.arange(pad, dtype=jnp.int32) % (N_OUT - N))
    dst = jnp.concatenate([edge_index[1], pad_dst]).reshape(E_PAD // CH, CH)
    partials = _sc_agg(z, src, dst)
    return _mlp(partials, W1, b1, W2, b2, W3, b3, W4, b4)
